# baseline (speedup 1.0000x reference)
import jax
import jax.numpy as jnp
from jax.experimental import pallas as pl
from jax.experimental.pallas import tpu as pltpu

D3 = 256
D_MODEL = 768
SEQ = 81
REP = 32


def _pe_build_kernel(rows_ref, cols_ref, boxes_ref,
                     row_tab_ref, col_tab_ref, box_tab_ref,
                     rpe_ref, cpe_ref, bpe_ref):
    iota = jax.lax.broadcasted_iota(jnp.int32, (SEQ, 9), 1)
    oh_rows = (rows_ref[...] == iota).astype(jnp.float32)
    oh_cols = (cols_ref[...] == iota).astype(jnp.float32)
    oh_boxes = (boxes_ref[...] == iota).astype(jnp.float32)
    rpe_ref[...] = jnp.dot(oh_rows, row_tab_ref[...],
                           preferred_element_type=jnp.float32)
    cpe_ref[...] = jnp.dot(oh_cols, col_tab_ref[...],
                           preferred_element_type=jnp.float32)
    bpe_ref[...] = jnp.dot(oh_boxes, box_tab_ref[...],
                           preferred_element_type=jnp.float32)


def _add_kernel(rpe_ref, cpe_ref, bpe_ref, x_ref, out_ref, pe_ref):
    @pl.when(pl.program_id(0) == 0)
    def _build_pe():
        pe_ref[:, 0:D3] = rpe_ref[...]
        pe_ref[:, D3:2 * D3] = cpe_ref[...]
        pe_ref[:, 2 * D3:D_MODEL] = bpe_ref[...]

    out_ref[...] = x_ref[...] + pe_ref[...][None, :, :]


@jax.jit
def kernel(x, row_table, col_table, box_table, rows, cols, boxes):
    pe_shape = jax.ShapeDtypeStruct((SEQ, D3), jnp.float32)
    row_pe, col_pe, box_pe = pl.pallas_call(
        _pe_build_kernel,
        out_shape=[pe_shape, pe_shape, pe_shape],
    )(rows.reshape(SEQ, 1), cols.reshape(SEQ, 1), boxes.reshape(SEQ, 1),
      row_table, col_table, box_table)

    b = x.shape[0]
    pe_spec = pl.BlockSpec((SEQ, D3), lambda i: (0, 0))
    return pl.pallas_call(
        _add_kernel,
        grid=(b // REP,),
        in_specs=[
            pe_spec, pe_spec, pe_spec,
            pl.BlockSpec((REP, SEQ, D_MODEL), lambda i: (i, 0, 0)),
        ],
        out_specs=pl.BlockSpec((REP, SEQ, D_MODEL), lambda i: (i, 0, 0)),
        out_shape=jax.ShapeDtypeStruct(x.shape, x.dtype),
        scratch_shapes=[pltpu.VMEM((SEQ, D_MODEL), jnp.float32)],
        compiler_params=pltpu.CompilerParams(
            dimension_semantics=("arbitrary",),
        ),
    )(row_pe, col_pe, box_pe, x)


# speedup vs baseline: 3.3495x; 3.3495x over previous
import jax
import jax.numpy as jnp
from jax.experimental import pallas as pl
from jax.experimental.pallas import tpu as pltpu

D3 = 256
D_MODEL = 768
SEQ = 81
BB = 2048


def _pe_build_kernel(rows_ref, cols_ref, boxes_ref,
                     row_tab_ref, col_tab_ref, box_tab_ref, pe_ref):
    iota = jax.lax.broadcasted_iota(jnp.int32, (SEQ, 9), 1)
    oh_rows = (rows_ref[...] == iota).astype(jnp.float32)
    oh_cols = (cols_ref[...] == iota).astype(jnp.float32)
    oh_boxes = (boxes_ref[...] == iota).astype(jnp.float32)
    pe_ref[:, 0:D3] = jnp.dot(oh_rows, row_tab_ref[...],
                              preferred_element_type=jnp.float32)
    pe_ref[:, D3:2 * D3] = jnp.dot(oh_cols, col_tab_ref[...],
                                   preferred_element_type=jnp.float32)
    pe_ref[:, 2 * D3:D_MODEL] = jnp.dot(oh_boxes, box_tab_ref[...],
                                        preferred_element_type=jnp.float32)


def _add_kernel(pe_ref, x_ref, out_ref):
    out_ref[...] = x_ref[...] + pe_ref[0]


@jax.jit
def kernel(x, row_table, col_table, box_table, rows, cols, boxes):
    pe = pl.pallas_call(
        _pe_build_kernel,
        out_shape=jax.ShapeDtypeStruct((SEQ, D_MODEL), jnp.float32),
    )(rows.reshape(SEQ, 1), cols.reshape(SEQ, 1), boxes.reshape(SEQ, 1),
      row_table, col_table, box_table)

    b = x.shape[0]
    blocks_per_cell = b // BB
    x2 = x.transpose(1, 0, 2).reshape(SEQ * b, D_MODEL)
    pe3 = pe.reshape(SEQ, 1, D_MODEL)
    out2 = pl.pallas_call(
        _add_kernel,
        grid=(SEQ * b // BB,),
        in_specs=[
            pl.BlockSpec((1, 1, D_MODEL),
                         lambda i: (i // blocks_per_cell, 0, 0)),
            pl.BlockSpec((BB, D_MODEL), lambda i: (i, 0)),
        ],
        out_specs=pl.BlockSpec((BB, D_MODEL), lambda i: (i, 0)),
        out_shape=jax.ShapeDtypeStruct(x2.shape, x2.dtype),
        compiler_params=pltpu.CompilerParams(
            dimension_semantics=("arbitrary",),
        ),
    )(pe3, x2)
    return out2.reshape(SEQ, b, D_MODEL).transpose(1, 0, 2)


# speedup vs baseline: 3.4031x; 1.0160x over previous
import jax
import jax.numpy as jnp
from jax.experimental import pallas as pl
from jax.experimental.pallas import tpu as pltpu

D3 = 256
D_MODEL = 768
SEQ = 81


def _pe_add_kernel(rows_ref, cols_ref, boxes_ref,
                   row_tab_ref, col_tab_ref, box_tab_ref,
                   x_ref, out_ref, pe_ref):
    @pl.when(pl.program_id(0) == 0)
    def _build_pe():
        for p in range(SEQ):
            r = rows_ref[p]
            c = cols_ref[p]
            bx = boxes_ref[p]
            pe_ref[pl.ds(p, 1), 0:D3] = row_tab_ref[pl.ds(r, 1), :]
            pe_ref[pl.ds(p, 1), D3:2 * D3] = col_tab_ref[pl.ds(c, 1), :]
            pe_ref[pl.ds(p, 1), 2 * D3:D_MODEL] = box_tab_ref[pl.ds(bx, 1), :]

    cell = pl.program_id(0)
    out_ref[...] = x_ref[...] + pe_ref[pl.ds(cell, 1), :]


@jax.jit
def kernel(x, row_table, col_table, box_table, rows, cols, boxes):
    b = x.shape[0]
    x2 = x.transpose(1, 0, 2).reshape(SEQ * b, D_MODEL)
    smem = pl.BlockSpec(memory_space=pltpu.SMEM)
    full = lambda shape: pl.BlockSpec(shape, lambda i: (0,) * len(shape))
    out2 = pl.pallas_call(
        _pe_add_kernel,
        grid=(SEQ,),
        in_specs=[
            smem, smem, smem,
            full((9, D3)),
            full((9, D3)),
            full((9, D_MODEL - 2 * D3)),
            pl.BlockSpec((b, D_MODEL), lambda i: (i, 0)),
        ],
        out_specs=pl.BlockSpec((b, D_MODEL), lambda i: (i, 0)),
        out_shape=jax.ShapeDtypeStruct(x2.shape, x2.dtype),
        scratch_shapes=[pltpu.VMEM((SEQ, D_MODEL), jnp.float32)],
        compiler_params=pltpu.CompilerParams(
            dimension_semantics=("arbitrary",),
        ),
    )(rows, cols, boxes, row_table, col_table, box_table, x2)
    return out2.reshape(SEQ, b, D_MODEL).transpose(1, 0, 2)
